# baseline (speedup 1.0000x reference)
import functools

import jax
import jax.numpy as jnp
from jax import lax
from jax.experimental import pallas as pl
from jax.experimental.pallas import tpu as pltpu
from jax.experimental.pallas import tpu_sc as plsc

N_NODES = 10000
N_EDGES = 320000
D = 128

NC = 2
NS = 16
NW = NC * NS
EW = N_EDGES // NW
EW_P = 10240
CHUNK = 128
NCHUNK = EW_P // CHUNK
N_PAD = 10240
PAD_ROW = N_NODES + 8
ROWS_PER_TILE = N_PAD // NS
NGROUP = NCHUNK // 4

_mesh = plsc.VectorSubcoreMesh(core_axis_name="c", subcore_axis_name="s")


def _idx_load(idx_hbm, ebase, j, slot, sem):
    return pltpu.async_copy(idx_hbm.at[pl.ds(ebase + j * CHUNK, CHUNK)], slot, sem)


def _idx_wait(idx_hbm, ebase, j, slot, sem):
    pltpu.make_async_copy(idx_hbm.at[pl.ds(ebase + j * CHUNK, CHUNK)], slot, sem).wait()


DEG_R = ROWS_PER_TILE // D
NKVEC = EW_P // 16


@functools.partial(
    pl.kernel,
    out_type=jax.ShapeDtypeStruct((NC, NS, DEG_R, D), jnp.float32),
    mesh=_mesh,
    compiler_params=pltpu.CompilerParams(needs_layout_passes=False),
    scratch_types=[
        pltpu.VMEM((EW_P,), jnp.int32),
        pltpu.VMEM((N_PAD,), jnp.float32),
        pltpu.VMEM((NS, ROWS_PER_TILE), jnp.float32),
        pltpu.VMEM((DEG_R, D), jnp.float32),
        pltpu.VMEM_SHARED((NS, N_PAD), jnp.float32),
    ],
)
def _sc_degree(dst_hbm, zeros1_hbm, out_hbm,
               dst_v, hist_v, stripes_v, acc2_v, hists_sh):
    cid = lax.axis_index("c")
    sid = lax.axis_index("s")
    wid = sid * NC + cid
    base = sid * ROWS_PER_TILE

    pltpu.sync_copy(dst_hbm.at[pl.ds(wid * EW_P, EW_P)], dst_v)
    pltpu.sync_copy(zeros1_hbm, hist_v)

    def body(k, _):
        iv = dst_v[pl.ds(k * 16, 16)]
        plsc.addupdate_scatter(hist_v, [iv], jnp.ones((16,), jnp.float32))
        return 0

    lax.fori_loop(0, NKVEC, body, 0)
    pltpu.sync_copy(hist_v, hists_sh.at[sid])
    plsc.subcore_barrier()
    pltpu.sync_copy(hists_sh.at[:, pl.ds(base, ROWS_PER_TILE)], stripes_v)
    for r in range(DEG_R):
        for c in range(D // 16):
            sl = pl.ds(r * D + c * 16, 16)
            s = stripes_v[0, sl]
            for h in range(1, NS):
                s = s + stripes_v[h, sl]
            acc2_v[r, pl.ds(c * 16, 16)] = s
    pltpu.sync_copy(acc2_v, out_hbm.at[cid, sid])


@functools.partial(
    pl.kernel,
    out_type=jax.ShapeDtypeStruct((NC, N_PAD, D), jnp.float32),
    mesh=_mesh,
    scratch_types=[
        pltpu.VMEM((CHUNK,), jnp.int32),
        pltpu.VMEM((CHUNK,), jnp.int32),
        pltpu.VMEM((CHUNK,), jnp.int32),
        pltpu.VMEM((CHUNK,), jnp.int32),
        pltpu.VMEM((CHUNK,), jnp.int32),
        pltpu.VMEM((CHUNK,), jnp.int32),
        pltpu.VMEM((CHUNK,), jnp.int32),
        pltpu.VMEM((CHUNK,), jnp.int32),
        pltpu.VMEM((CHUNK, D), jnp.float32),
        pltpu.VMEM((CHUNK, D), jnp.float32),
        pltpu.VMEM_SHARED((N_PAD, D), jnp.float32),
        pltpu.SemaphoreType.DMA,
        pltpu.SemaphoreType.DMA,
        pltpu.SemaphoreType.DMA,
    ],
)
def _sc_edge_agg(g_hbm, src_hbm, dst_hbm, zeros_hbm, out_hbm,
                 s0, s1, s2, s3, d0, d1, d2, d3, r0, r1,
                 acc_sh, isem, gsem, ssem):
    cid = lax.axis_index("c")
    sid = lax.axis_index("s")
    wid = sid * NC + cid
    ebase = wid * EW_P
    base = sid * ROWS_PER_TILE
    sslot = [s0, s1, s2, s3]
    dslot = [d0, d1, d2, d3]
    rows = [r0, r1]

    pltpu.sync_copy(zeros_hbm.at[pl.ds(base, ROWS_PER_TILE)],
                    acc_sh.at[pl.ds(base, ROWS_PER_TILE)])
    pltpu.sync_copy(src_hbm.at[pl.ds(ebase, CHUNK)], s0)
    pltpu.sync_copy(dst_hbm.at[pl.ds(ebase, CHUNK)], d0)
    _idx_load(src_hbm, ebase, 1, s1, isem)
    _idx_load(dst_hbm, ebase, 1, d1, isem)
    plsc.subcore_barrier()

    pltpu.async_copy(g_hbm.at[s0], r0, gsem)

    def body(g, _):
        for r in range(4):
            j = 4 * g + r

            @pl.when(j >= 1)
            def _():
                pltpu.make_async_copy(rows[(r + 1) % 2],
                                      acc_sh.at[dslot[(r + 3) % 4]],
                                      ssem).wait()

            @pl.when(j + 1 < NCHUNK)
            def _():
                _idx_wait(src_hbm, ebase, j + 1, sslot[(r + 1) % 4], isem)
                _idx_wait(dst_hbm, ebase, j + 1, dslot[(r + 1) % 4], isem)
                pltpu.async_copy(g_hbm.at[sslot[(r + 1) % 4]],
                                 rows[(r + 1) % 2], gsem)

            @pl.when(j + 2 < NCHUNK)
            def _():
                _idx_load(src_hbm, ebase, j + 2, sslot[(r + 2) % 4], isem)
                _idx_load(dst_hbm, ebase, j + 2, dslot[(r + 2) % 4], isem)

            pltpu.make_async_copy(g_hbm.at[sslot[r]], rows[r % 2], gsem).wait()
            pltpu.async_copy(rows[r % 2], acc_sh.at[dslot[r]], ssem, add=True)
        return 0

    lax.fori_loop(0, NGROUP, body, 0)
    pltpu.make_async_copy(r1, acc_sh.at[d3], ssem).wait()
    plsc.subcore_barrier()
    pltpu.sync_copy(acc_sh.at[pl.ds(base, ROWS_PER_TILE)],
                    out_hbm.at[cid, pl.ds(base, ROWS_PER_TILE)])


BR = 1000
GRID = N_NODES // BR


def _dinv_from_parts(deg_ref):
    deg = deg_ref[0, :, 0] + deg_ref[1, :, 0] + 1.0
    return lax.rsqrt(deg)[:, None]


def _tc_pre_body(deg_ref, x_ref, w_ref, g_ref):
    dinv = _dinv_from_parts(deg_ref)
    g_ref[...] = jnp.dot(x_ref[...], w_ref[...],
                         preferred_element_type=jnp.float32) * dinv


def _tc_mid_body(deg_ref, s_ref, g_ref, b_ref, w_ref, g2_ref):
    dinv = _dinv_from_parts(deg_ref)
    h = jnp.maximum(dinv * (s_ref[0] + s_ref[1] + g_ref[...]) + b_ref[...], 0.0)
    g2_ref[...] = jnp.dot(h, w_ref[...],
                          preferred_element_type=jnp.float32) * dinv


def _tc_post_body(deg_ref, s_ref, g_ref, b_ref, out_ref):
    dinv = _dinv_from_parts(deg_ref)
    out_ref[...] = dinv * (s_ref[0] + s_ref[1] + g_ref[...]) + b_ref[...]


_deg_spec = pl.BlockSpec((NC, BR, 1), lambda i: (0, i, 0))
_row_spec = pl.BlockSpec((BR, D), lambda i: (i, 0))
_parts_spec = pl.BlockSpec((NC, BR, D), lambda i: (0, i, 0))
_mat_spec = pl.BlockSpec((D, D), lambda i: (0, 0))
_vec_spec = pl.BlockSpec((1, D), lambda i: (0, 0))

_tc_pre = pl.pallas_call(
    _tc_pre_body,
    grid=(GRID,),
    in_specs=[_deg_spec, _row_spec, _mat_spec],
    out_specs=_row_spec,
    out_shape=jax.ShapeDtypeStruct((N_NODES, D), jnp.float32),
)

_tc_mid = pl.pallas_call(
    _tc_mid_body,
    grid=(GRID,),
    in_specs=[_deg_spec, _parts_spec, _row_spec, _vec_spec, _mat_spec],
    out_specs=_row_spec,
    out_shape=jax.ShapeDtypeStruct((N_NODES, D), jnp.float32),
)

_tc_post = pl.pallas_call(
    _tc_post_body,
    grid=(GRID,),
    in_specs=[_deg_spec, _parts_spec, _row_spec, _vec_spec],
    out_specs=_row_spec,
    out_shape=jax.ShapeDtypeStruct((N_NODES, D), jnp.float32),
)


def kernel(x, edge_index, W1, b1, W2, b2):
    pad = ((0, 0), (0, EW_P - EW))
    src = jnp.pad(edge_index[0].astype(jnp.int32).reshape(NW, EW),
                  pad).reshape(-1)
    dst = jnp.pad(edge_index[1].astype(jnp.int32).reshape(NW, EW),
                  pad, constant_values=PAD_ROW).reshape(-1)
    b1r = b1.reshape(1, D)
    b2r = b2.reshape(1, D)
    zeros = jnp.zeros((N_PAD, D), jnp.float32)

    degp = _sc_degree(dst, jnp.zeros((N_PAD,), jnp.float32))
    deg_parts = degp.reshape(NC, N_PAD)[:, :N_NODES].reshape(NC, N_NODES, 1)
    g1 = _tc_pre(deg_parts, x, W1)
    s1 = _sc_edge_agg(g1, src, dst, zeros)
    g2 = _tc_mid(deg_parts, s1, g1, b1r, W2)
    s2 = _sc_edge_agg(g2, src, dst, zeros)
    return _tc_post(deg_parts, s2, g2, b2r)


# speedup vs baseline: 2.6901x; 2.6901x over previous
import functools

import jax
import jax.numpy as jnp
from jax import lax
from jax.experimental import pallas as pl
from jax.experimental.pallas import tpu as pltpu
from jax.experimental.pallas import tpu_sc as plsc

N_NODES = 10000
N_EDGES = 320000
D = 128

NC = 2
NS = 16
NW = NC * NS
EW = N_EDGES // NW
CHUNK = 80
NCHUNK = EW // CHUNK
N_PAD = 10240
ROWS_PER_TILE = N_PAD // NS
NGROUP = (NCHUNK - 1) // 4

_mesh = plsc.VectorSubcoreMesh(core_axis_name="c", subcore_axis_name="s")


def _idx_load(idx_hbm, ebase, j, slot, sem):
    return pltpu.async_copy(idx_hbm.at[pl.ds(ebase + j * CHUNK, CHUNK)], slot, sem)


def _idx_wait(idx_hbm, ebase, j, slot, sem):
    pltpu.make_async_copy(idx_hbm.at[pl.ds(ebase + j * CHUNK, CHUNK)], slot, sem).wait()


DEG_R = ROWS_PER_TILE // D
NKVEC = EW // 16


@functools.partial(
    pl.kernel,
    out_type=jax.ShapeDtypeStruct((NC, NS, DEG_R, D), jnp.float32),
    mesh=_mesh,
    compiler_params=pltpu.CompilerParams(needs_layout_passes=False),
    scratch_types=[
        pltpu.VMEM((EW,), jnp.int32),
        pltpu.VMEM((N_PAD,), jnp.float32),
        pltpu.VMEM((NS, ROWS_PER_TILE), jnp.float32),
        pltpu.VMEM((DEG_R, D), jnp.float32),
        pltpu.VMEM_SHARED((NS, N_PAD), jnp.float32),
    ],
)
def _sc_degree(dst_hbm, zeros1_hbm, out_hbm,
               dst_v, hist_v, stripes_v, acc2_v, hists_sh):
    cid = lax.axis_index("c")
    sid = lax.axis_index("s")
    wid = sid * NC + cid
    base = sid * ROWS_PER_TILE

    pltpu.sync_copy(dst_hbm.at[pl.ds(wid * EW, EW)], dst_v)
    pltpu.sync_copy(zeros1_hbm, hist_v)

    def body(k, _):
        iv = dst_v[pl.ds(k * 16, 16)]
        plsc.addupdate_scatter(hist_v, [iv], jnp.ones((16,), jnp.float32))
        return 0

    lax.fori_loop(0, NKVEC, body, 0)
    pltpu.sync_copy(hist_v, hists_sh.at[sid])
    plsc.subcore_barrier()
    pltpu.sync_copy(hists_sh.at[:, pl.ds(base, ROWS_PER_TILE)], stripes_v)
    for r in range(DEG_R):
        for c in range(D // 16):
            sl = pl.ds(r * D + c * 16, 16)
            s = stripes_v[0, sl]
            for h in range(1, NS):
                s = s + stripes_v[h, sl]
            acc2_v[r, pl.ds(c * 16, 16)] = s
    pltpu.sync_copy(acc2_v, out_hbm.at[cid, sid])


@functools.partial(
    pl.kernel,
    out_type=jax.ShapeDtypeStruct((NC, N_PAD, D), jnp.float32),
    mesh=_mesh,
    scratch_types=[
        pltpu.VMEM((CHUNK,), jnp.int32),
        pltpu.VMEM((CHUNK,), jnp.int32),
        pltpu.VMEM((CHUNK,), jnp.int32),
        pltpu.VMEM((CHUNK,), jnp.int32),
        pltpu.VMEM((CHUNK,), jnp.int32),
        pltpu.VMEM((CHUNK,), jnp.int32),
        pltpu.VMEM((CHUNK,), jnp.int32),
        pltpu.VMEM((CHUNK,), jnp.int32),
        pltpu.VMEM((CHUNK, D), jnp.float32),
        pltpu.VMEM((CHUNK, D), jnp.float32),
        pltpu.VMEM_SHARED((N_PAD, D), jnp.float32),
        pltpu.SemaphoreType.DMA,
        pltpu.SemaphoreType.DMA,
        pltpu.SemaphoreType.DMA,
    ],
)
def _sc_edge_agg(g_hbm, src_hbm, dst_hbm, zeros_hbm, out_hbm,
                 s0, s1, s2, s3, d0, d1, d2, d3, r0, r1,
                 acc_sh, isem, gsem, ssem):
    cid = lax.axis_index("c")
    sid = lax.axis_index("s")
    wid = sid * NC + cid
    ebase = wid * EW
    base = sid * ROWS_PER_TILE
    sslot = [s0, s1, s2, s3]
    dslot = [d0, d1, d2, d3]
    rows = [r0, r1]

    pltpu.sync_copy(zeros_hbm.at[pl.ds(base, ROWS_PER_TILE)],
                    acc_sh.at[pl.ds(base, ROWS_PER_TILE)])
    pltpu.sync_copy(src_hbm.at[pl.ds(ebase, CHUNK)], s0)
    pltpu.sync_copy(dst_hbm.at[pl.ds(ebase, CHUNK)], d0)
    _idx_load(src_hbm, ebase, 1, s1, isem)
    _idx_load(dst_hbm, ebase, 1, d1, isem)
    plsc.subcore_barrier()

    pltpu.async_copy(g_hbm.at[s0], r0, gsem)

    def body(g, _):
        for r in range(4):
            j = 4 * g + r

            @pl.when(j >= 1)
            def _():
                pltpu.make_async_copy(rows[(r + 1) % 2],
                                      acc_sh.at[dslot[(r + 3) % 4]],
                                      ssem).wait()

            @pl.when(j + 1 < NCHUNK)
            def _():
                _idx_wait(src_hbm, ebase, j + 1, sslot[(r + 1) % 4], isem)
                _idx_wait(dst_hbm, ebase, j + 1, dslot[(r + 1) % 4], isem)
                pltpu.async_copy(g_hbm.at[sslot[(r + 1) % 4]],
                                 rows[(r + 1) % 2], gsem)

            @pl.when(j + 2 < NCHUNK)
            def _():
                _idx_load(src_hbm, ebase, j + 2, sslot[(r + 2) % 4], isem)
                _idx_load(dst_hbm, ebase, j + 2, dslot[(r + 2) % 4], isem)

            pltpu.make_async_copy(g_hbm.at[sslot[r]], rows[r % 2], gsem).wait()
            pltpu.async_copy(rows[r % 2], acc_sh.at[dslot[r]], ssem, add=True)
        return 0

    lax.fori_loop(0, NGROUP, body, 0)
    pltpu.make_async_copy(r1, acc_sh.at[d3], ssem).wait()
    pltpu.make_async_copy(g_hbm.at[s0], r0, gsem).wait()
    pltpu.async_copy(r0, acc_sh.at[d0], ssem, add=True)
    pltpu.make_async_copy(r0, acc_sh.at[d0], ssem).wait()
    plsc.subcore_barrier()
    pltpu.sync_copy(acc_sh.at[pl.ds(base, ROWS_PER_TILE)],
                    out_hbm.at[cid, pl.ds(base, ROWS_PER_TILE)])


BR = 1000
GRID = N_NODES // BR


def _dinv_from_parts(deg_ref):
    deg = deg_ref[0, :, 0] + deg_ref[1, :, 0] + 1.0
    return lax.rsqrt(deg)[:, None]


def _tc_pre_body(deg_ref, x_ref, w_ref, g_ref):
    dinv = _dinv_from_parts(deg_ref)
    g_ref[...] = jnp.dot(x_ref[...], w_ref[...],
                         preferred_element_type=jnp.float32) * dinv


def _tc_mid_body(deg_ref, s_ref, g_ref, b_ref, w_ref, g2_ref):
    dinv = _dinv_from_parts(deg_ref)
    h = jnp.maximum(dinv * (s_ref[0] + s_ref[1] + g_ref[...]) + b_ref[...], 0.0)
    g2_ref[...] = jnp.dot(h, w_ref[...],
                          preferred_element_type=jnp.float32) * dinv


def _tc_post_body(deg_ref, s_ref, g_ref, b_ref, out_ref):
    dinv = _dinv_from_parts(deg_ref)
    out_ref[...] = dinv * (s_ref[0] + s_ref[1] + g_ref[...]) + b_ref[...]


_deg_spec = pl.BlockSpec((NC, BR, 1), lambda i: (0, i, 0))
_row_spec = pl.BlockSpec((BR, D), lambda i: (i, 0))
_parts_spec = pl.BlockSpec((NC, BR, D), lambda i: (0, i, 0))
_mat_spec = pl.BlockSpec((D, D), lambda i: (0, 0))
_vec_spec = pl.BlockSpec((1, D), lambda i: (0, 0))

_tc_pre = pl.pallas_call(
    _tc_pre_body,
    grid=(GRID,),
    in_specs=[_deg_spec, _row_spec, _mat_spec],
    out_specs=_row_spec,
    out_shape=jax.ShapeDtypeStruct((N_NODES, D), jnp.float32),
)

_tc_mid = pl.pallas_call(
    _tc_mid_body,
    grid=(GRID,),
    in_specs=[_deg_spec, _parts_spec, _row_spec, _vec_spec, _mat_spec],
    out_specs=_row_spec,
    out_shape=jax.ShapeDtypeStruct((N_NODES, D), jnp.float32),
)

_tc_post = pl.pallas_call(
    _tc_post_body,
    grid=(GRID,),
    in_specs=[_deg_spec, _parts_spec, _row_spec, _vec_spec],
    out_specs=_row_spec,
    out_shape=jax.ShapeDtypeStruct((N_NODES, D), jnp.float32),
)


def kernel(x, edge_index, W1, b1, W2, b2):
    src = edge_index[0].astype(jnp.int32)
    dst = edge_index[1].astype(jnp.int32)
    b1r = b1.reshape(1, D)
    b2r = b2.reshape(1, D)
    zeros = jnp.zeros((N_PAD, D), jnp.float32)

    degp = _sc_degree(dst, jnp.zeros((N_PAD,), jnp.float32))
    deg_parts = degp.reshape(NC, N_PAD)[:, :N_NODES].reshape(NC, N_NODES, 1)
    g1 = _tc_pre(deg_parts, x, W1)
    s1 = _sc_edge_agg(g1, src, dst, zeros)
    g2 = _tc_mid(deg_parts, s1, g1, b1r, W2)
    s2 = _sc_edge_agg(g2, src, dst, zeros)
    return _tc_post(deg_parts, s2, g2, b2r)


# speedup vs baseline: 2.7436x; 1.0199x over previous
import functools

import jax
import jax.numpy as jnp
from jax import lax
from jax.experimental import pallas as pl
from jax.experimental.pallas import tpu as pltpu
from jax.experimental.pallas import tpu_sc as plsc

N_NODES = 10000
N_EDGES = 320000
D = 128

NC = 2
NS = 16
NW = NC * NS
EW = N_EDGES // NW
CHUNK = 80
NCHUNK = EW // CHUNK
N_PAD = 10240
ROWS_PER_TILE = N_PAD // NS
NGROUP = (NCHUNK - 1) // 4

_mesh = plsc.VectorSubcoreMesh(core_axis_name="c", subcore_axis_name="s")


def _idx_load(idx_hbm, ebase, j, slot, sem):
    return pltpu.async_copy(idx_hbm.at[pl.ds(ebase + j * CHUNK, CHUNK)], slot, sem)


def _idx_wait(idx_hbm, ebase, j, slot, sem):
    pltpu.make_async_copy(idx_hbm.at[pl.ds(ebase + j * CHUNK, CHUNK)], slot, sem).wait()


DEG_R = ROWS_PER_TILE // D
NKVEC = EW // 16


@functools.partial(
    pl.kernel,
    out_type=jax.ShapeDtypeStruct((NC, NS, DEG_R, D), jnp.float32),
    mesh=_mesh,
    compiler_params=pltpu.CompilerParams(needs_layout_passes=False),
    scratch_types=[
        pltpu.VMEM((EW,), jnp.int32),
        pltpu.VMEM((N_PAD,), jnp.float32),
        pltpu.VMEM((NS, ROWS_PER_TILE), jnp.float32),
        pltpu.VMEM((DEG_R, D), jnp.float32),
        pltpu.VMEM_SHARED((NS, N_PAD), jnp.float32),
    ],
)
def _sc_degree(dst_hbm, zeros1_hbm, out_hbm,
               dst_v, hist_v, stripes_v, acc2_v, hists_sh):
    cid = lax.axis_index("c")
    sid = lax.axis_index("s")
    wid = sid * NC + cid
    base = sid * ROWS_PER_TILE

    pltpu.sync_copy(dst_hbm.at[pl.ds(wid * EW, EW)], dst_v)
    pltpu.sync_copy(zeros1_hbm, hist_v)

    def body(k, _):
        iv = dst_v[pl.ds(k * 16, 16)]
        plsc.addupdate_scatter(hist_v, [iv], jnp.ones((16,), jnp.float32))
        return 0

    lax.fori_loop(0, NKVEC, body, 0)
    pltpu.sync_copy(hist_v, hists_sh.at[sid])
    plsc.subcore_barrier()
    pltpu.sync_copy(hists_sh.at[:, pl.ds(base, ROWS_PER_TILE)], stripes_v)
    for r in range(DEG_R):
        for c in range(D // 16):
            sl = pl.ds(r * D + c * 16, 16)
            s = stripes_v[0, sl]
            for h in range(1, NS):
                s = s + stripes_v[h, sl]
            acc2_v[r, pl.ds(c * 16, 16)] = s
    pltpu.sync_copy(acc2_v, out_hbm.at[cid, sid])


@functools.partial(
    pl.kernel,
    out_type=jax.ShapeDtypeStruct((NC, N_PAD, D), jnp.float32),
    mesh=_mesh,
    scratch_types=[
        pltpu.VMEM((CHUNK,), jnp.int32),
        pltpu.VMEM((CHUNK,), jnp.int32),
        pltpu.VMEM((CHUNK,), jnp.int32),
        pltpu.VMEM((CHUNK,), jnp.int32),
        pltpu.VMEM((CHUNK,), jnp.int32),
        pltpu.VMEM((CHUNK,), jnp.int32),
        pltpu.VMEM((CHUNK,), jnp.int32),
        pltpu.VMEM((CHUNK,), jnp.int32),
        pltpu.VMEM((CHUNK, D), jnp.float32),
        pltpu.VMEM((CHUNK, D), jnp.float32),
        pltpu.VMEM_SHARED((N_PAD, D), jnp.float32),
        pltpu.SemaphoreType.DMA,
        pltpu.SemaphoreType.DMA,
        pltpu.SemaphoreType.DMA,
    ],
)
def _sc_edge_agg(g_hbm, src_hbm, dst_hbm, zeros_hbm, out_hbm,
                 s0, s1, s2, s3, d0, d1, d2, d3, r0, r1,
                 acc_sh, isem, gsem, ssem):
    cid = lax.axis_index("c")
    sid = lax.axis_index("s")
    wid = sid * NC + cid
    ebase = wid * EW
    base = sid * ROWS_PER_TILE
    sslot = [s0, s1, s2, s3]
    dslot = [d0, d1, d2, d3]
    rows = [r0, r1]

    pltpu.sync_copy(zeros_hbm.at[pl.ds(base, ROWS_PER_TILE)],
                    acc_sh.at[pl.ds(base, ROWS_PER_TILE)])
    pltpu.sync_copy(src_hbm.at[pl.ds(ebase, CHUNK)], s0)
    pltpu.sync_copy(dst_hbm.at[pl.ds(ebase, CHUNK)], d0)
    _idx_load(src_hbm, ebase, 1, s1, isem)
    _idx_load(dst_hbm, ebase, 1, d1, isem)
    plsc.subcore_barrier()

    pltpu.async_copy(g_hbm.at[s0], r0, gsem)

    def body(g, _):
        for r in range(4):
            j = 4 * g + r

            @pl.when(j >= 1)
            def _():
                pltpu.make_async_copy(rows[(r + 1) % 2],
                                      acc_sh.at[dslot[(r + 3) % 4]],
                                      ssem).wait()

            @pl.when(j + 1 < NCHUNK)
            def _():
                _idx_wait(src_hbm, ebase, j + 1, sslot[(r + 1) % 4], isem)
                _idx_wait(dst_hbm, ebase, j + 1, dslot[(r + 1) % 4], isem)
                pltpu.async_copy(g_hbm.at[sslot[(r + 1) % 4]],
                                 rows[(r + 1) % 2], gsem)

            @pl.when(j + 2 < NCHUNK)
            def _():
                _idx_load(src_hbm, ebase, j + 2, sslot[(r + 2) % 4], isem)
                _idx_load(dst_hbm, ebase, j + 2, dslot[(r + 2) % 4], isem)

            pltpu.make_async_copy(g_hbm.at[sslot[r]], rows[r % 2], gsem).wait()
            pltpu.async_copy(rows[r % 2], acc_sh.at[dslot[r]], ssem, add=True)
        return 0

    lax.fori_loop(0, NGROUP, body, 0)
    pltpu.make_async_copy(r1, acc_sh.at[d3], ssem).wait()
    pltpu.make_async_copy(g_hbm.at[s0], r0, gsem).wait()
    pltpu.async_copy(r0, acc_sh.at[d0], ssem, add=True)
    pltpu.make_async_copy(r0, acc_sh.at[d0], ssem).wait()
    plsc.subcore_barrier()
    pltpu.sync_copy(acc_sh.at[pl.ds(base, ROWS_PER_TILE)],
                    out_hbm.at[cid, pl.ds(base, ROWS_PER_TILE)])


BR = 2000
GRID = N_NODES // BR


def _dinv_from_parts(deg_ref):
    deg = deg_ref[0, :, 0] + deg_ref[1, :, 0] + 1.0
    return lax.rsqrt(deg)[:, None]


def _tc_pre_body(deg_ref, x_ref, w_ref, g_ref):
    dinv = _dinv_from_parts(deg_ref)
    g_ref[...] = jnp.dot(x_ref[...], w_ref[...],
                         preferred_element_type=jnp.float32) * dinv


def _tc_mid_body(deg_ref, s_ref, g_ref, b_ref, w_ref, g2_ref):
    dinv = _dinv_from_parts(deg_ref)
    h = jnp.maximum(dinv * (s_ref[0] + s_ref[1] + g_ref[...]) + b_ref[...], 0.0)
    g2_ref[...] = jnp.dot(h, w_ref[...],
                          preferred_element_type=jnp.float32) * dinv


def _tc_post_body(deg_ref, s_ref, g_ref, b_ref, out_ref):
    dinv = _dinv_from_parts(deg_ref)
    out_ref[...] = dinv * (s_ref[0] + s_ref[1] + g_ref[...]) + b_ref[...]


_deg_spec = pl.BlockSpec((NC, BR, 1), lambda i: (0, i, 0))
_row_spec = pl.BlockSpec((BR, D), lambda i: (i, 0))
_parts_spec = pl.BlockSpec((NC, BR, D), lambda i: (0, i, 0))
_mat_spec = pl.BlockSpec((D, D), lambda i: (0, 0))
_vec_spec = pl.BlockSpec((1, D), lambda i: (0, 0))

_tc_pre = pl.pallas_call(
    _tc_pre_body,
    grid=(GRID,),
    in_specs=[_deg_spec, _row_spec, _mat_spec],
    out_specs=_row_spec,
    out_shape=jax.ShapeDtypeStruct((N_NODES, D), jnp.float32),
)

_tc_mid = pl.pallas_call(
    _tc_mid_body,
    grid=(GRID,),
    in_specs=[_deg_spec, _parts_spec, _row_spec, _vec_spec, _mat_spec],
    out_specs=_row_spec,
    out_shape=jax.ShapeDtypeStruct((N_NODES, D), jnp.float32),
)

_tc_post = pl.pallas_call(
    _tc_post_body,
    grid=(GRID,),
    in_specs=[_deg_spec, _parts_spec, _row_spec, _vec_spec],
    out_specs=_row_spec,
    out_shape=jax.ShapeDtypeStruct((N_NODES, D), jnp.float32),
)


def kernel(x, edge_index, W1, b1, W2, b2):
    src = edge_index[0].astype(jnp.int32)
    dst = edge_index[1].astype(jnp.int32)
    b1r = b1.reshape(1, D)
    b2r = b2.reshape(1, D)
    zeros = jnp.zeros((N_PAD, D), jnp.float32)

    degp = _sc_degree(dst, jnp.zeros((N_PAD,), jnp.float32))
    deg_parts = degp.reshape(NC, N_PAD)[:, :N_NODES].reshape(NC, N_NODES, 1)
    g1 = _tc_pre(deg_parts, x, W1)
    s1 = _sc_edge_agg(g1, src, dst, zeros)
    g2 = _tc_mid(deg_parts, s1, g1, b1r, W2)
    s2 = _sc_edge_agg(g2, src, dst, zeros)
    return _tc_post(deg_parts, s2, g2, b2r)


# speedup vs baseline: 3.1020x; 1.1307x over previous
import functools

import jax
import jax.numpy as jnp
from jax import lax
from jax.experimental import pallas as pl
from jax.experimental.pallas import tpu as pltpu
from jax.experimental.pallas import tpu_sc as plsc

N_NODES = 10000
N_EDGES = 320000
D = 128

NC = 2
NS = 16
NW = NC * NS
EW = N_EDGES // NW
CHUNK = 80
NCHUNK = EW // CHUNK
N_PAD = 10240
ROWS_PER_TILE = N_PAD // NS
NGROUP = (NCHUNK - 1) // 4

_mesh = plsc.VectorSubcoreMesh(core_axis_name="c", subcore_axis_name="s")


def _idx_load(idx_hbm, ebase, j, slot, sem):
    return pltpu.async_copy(idx_hbm.at[pl.ds(ebase + j * CHUNK, CHUNK)], slot, sem)


def _idx_wait(idx_hbm, ebase, j, slot, sem):
    pltpu.make_async_copy(idx_hbm.at[pl.ds(ebase + j * CHUNK, CHUNK)], slot, sem).wait()


DEG_R = ROWS_PER_TILE // D
NKVEC = EW // 16


@functools.partial(
    pl.kernel,
    out_type=jax.ShapeDtypeStruct((NC, NS, DEG_R, D), jnp.float32),
    mesh=_mesh,
    compiler_params=pltpu.CompilerParams(needs_layout_passes=False),
    scratch_types=[
        pltpu.VMEM((EW,), jnp.int32),
        pltpu.VMEM((N_PAD,), jnp.float32),
        pltpu.VMEM((NS, ROWS_PER_TILE), jnp.float32),
        pltpu.VMEM((DEG_R, D), jnp.float32),
        pltpu.VMEM_SHARED((NS, N_PAD), jnp.float32),
    ],
)
def _sc_degree(dst_hbm, zeros1_hbm, out_hbm,
               dst_v, hist_v, stripes_v, acc2_v, hists_sh):
    cid = lax.axis_index("c")
    sid = lax.axis_index("s")
    wid = sid * NC + cid
    base = sid * ROWS_PER_TILE

    pltpu.sync_copy(dst_hbm.at[pl.ds(wid * EW, EW)], dst_v)
    pltpu.sync_copy(zeros1_hbm, hist_v)

    def body(k, _):
        iv = dst_v[pl.ds(k * 16, 16)]
        plsc.addupdate_scatter(hist_v, [iv], jnp.ones((16,), jnp.float32))
        return 0

    lax.fori_loop(0, NKVEC, body, 0)
    pltpu.sync_copy(hist_v, hists_sh.at[sid])
    plsc.subcore_barrier()
    pltpu.sync_copy(hists_sh.at[:, pl.ds(base, ROWS_PER_TILE)], stripes_v)
    for r in range(DEG_R):
        for c in range(D // 16):
            sl = pl.ds(r * D + c * 16, 16)
            s = stripes_v[0, sl]
            for h in range(1, NS):
                s = s + stripes_v[h, sl]
            acc2_v[r, pl.ds(c * 16, 16)] = s
    pltpu.sync_copy(acc2_v, out_hbm.at[cid, sid])


@functools.partial(
    pl.kernel,
    out_type=jax.ShapeDtypeStruct((NC, N_PAD, D), jnp.float32),
    mesh=_mesh,
    scratch_types=[
        pltpu.VMEM((CHUNK,), jnp.int32),
        pltpu.VMEM((CHUNK,), jnp.int32),
        pltpu.VMEM((CHUNK,), jnp.int32),
        pltpu.VMEM((CHUNK,), jnp.int32),
        pltpu.VMEM((CHUNK,), jnp.int32),
        pltpu.VMEM((CHUNK,), jnp.int32),
        pltpu.VMEM((CHUNK,), jnp.int32),
        pltpu.VMEM((CHUNK,), jnp.int32),
        pltpu.VMEM((CHUNK, D), jnp.float32),
        pltpu.VMEM((CHUNK, D), jnp.float32),
        pltpu.VMEM((CHUNK, D), jnp.float32),
        pltpu.VMEM((CHUNK, D), jnp.float32),
        pltpu.VMEM_SHARED((N_PAD, D), jnp.float32),
        pltpu.SemaphoreType.DMA,
        pltpu.SemaphoreType.DMA,
        pltpu.SemaphoreType.DMA,
    ],
)
def _sc_edge_agg(g_hbm, src_hbm, dst_hbm, zeros_hbm, out_hbm,
                 s0, s1, s2, s3, d0, d1, d2, d3, r0, r1, r2, r3,
                 acc_sh, isem, gsem, ssem):
    cid = lax.axis_index("c")
    sid = lax.axis_index("s")
    wid = sid * NC + cid
    ebase = wid * EW
    base = sid * ROWS_PER_TILE
    sslot = [s0, s1, s2, s3]
    dslot = [d0, d1, d2, d3]
    rows = [r0, r1, r2, r3]

    pltpu.sync_copy(zeros_hbm.at[pl.ds(base, ROWS_PER_TILE)],
                    acc_sh.at[pl.ds(base, ROWS_PER_TILE)])
    pltpu.sync_copy(src_hbm.at[pl.ds(ebase, CHUNK)], s0)
    pltpu.sync_copy(dst_hbm.at[pl.ds(ebase, CHUNK)], d0)
    _idx_load(src_hbm, ebase, 1, s1, isem)
    _idx_load(dst_hbm, ebase, 1, d1, isem)
    plsc.subcore_barrier()

    pltpu.async_copy(g_hbm.at[s0], r0, gsem)

    def body(g, _):
        for r in range(4):
            j = 4 * g + r

            @pl.when(j >= 2)
            def _():
                pltpu.make_async_copy(rows[(r + 2) % 4],
                                      acc_sh.at[dslot[(r + 2) % 4]],
                                      ssem).wait()

            @pl.when(j + 1 < NCHUNK)
            def _():
                _idx_wait(src_hbm, ebase, j + 1, sslot[(r + 1) % 4], isem)
                _idx_wait(dst_hbm, ebase, j + 1, dslot[(r + 1) % 4], isem)
                pltpu.async_copy(g_hbm.at[sslot[(r + 1) % 4]],
                                 rows[(r + 1) % 4], gsem)

            @pl.when(j + 2 < NCHUNK)
            def _():
                _idx_load(src_hbm, ebase, j + 2, sslot[(r + 2) % 4], isem)
                _idx_load(dst_hbm, ebase, j + 2, dslot[(r + 2) % 4], isem)

            pltpu.make_async_copy(g_hbm.at[sslot[r]], rows[r], gsem).wait()
            pltpu.async_copy(rows[r], acc_sh.at[dslot[r]], ssem, add=True)
        return 0

    lax.fori_loop(0, NGROUP, body, 0)
    pltpu.make_async_copy(r2, acc_sh.at[d2], ssem).wait()
    pltpu.make_async_copy(g_hbm.at[s0], r0, gsem).wait()
    pltpu.async_copy(r0, acc_sh.at[d0], ssem, add=True)
    pltpu.make_async_copy(r3, acc_sh.at[d3], ssem).wait()
    pltpu.make_async_copy(r0, acc_sh.at[d0], ssem).wait()
    plsc.subcore_barrier()
    pltpu.sync_copy(acc_sh.at[pl.ds(base, ROWS_PER_TILE)],
                    out_hbm.at[cid, pl.ds(base, ROWS_PER_TILE)])


BR = 2000
GRID = N_NODES // BR


def _dinv_from_parts(deg_ref):
    deg = deg_ref[0, :, 0] + deg_ref[1, :, 0] + 1.0
    return lax.rsqrt(deg)[:, None]


def _tc_pre_body(deg_ref, x_ref, w_ref, g_ref):
    dinv = _dinv_from_parts(deg_ref)
    g_ref[...] = jnp.dot(x_ref[...], w_ref[...],
                         preferred_element_type=jnp.float32) * dinv


def _tc_mid_body(deg_ref, s_ref, g_ref, b_ref, w_ref, g2_ref):
    dinv = _dinv_from_parts(deg_ref)
    h = jnp.maximum(dinv * (s_ref[0] + s_ref[1] + g_ref[...]) + b_ref[...], 0.0)
    g2_ref[...] = jnp.dot(h, w_ref[...],
                          preferred_element_type=jnp.float32) * dinv


def _tc_post_body(deg_ref, s_ref, g_ref, b_ref, out_ref):
    dinv = _dinv_from_parts(deg_ref)
    out_ref[...] = dinv * (s_ref[0] + s_ref[1] + g_ref[...]) + b_ref[...]


_deg_spec = pl.BlockSpec((NC, BR, 1), lambda i: (0, i, 0))
_row_spec = pl.BlockSpec((BR, D), lambda i: (i, 0))
_parts_spec = pl.BlockSpec((NC, BR, D), lambda i: (0, i, 0))
_mat_spec = pl.BlockSpec((D, D), lambda i: (0, 0))
_vec_spec = pl.BlockSpec((1, D), lambda i: (0, 0))

_tc_pre = pl.pallas_call(
    _tc_pre_body,
    grid=(GRID,),
    in_specs=[_deg_spec, _row_spec, _mat_spec],
    out_specs=_row_spec,
    out_shape=jax.ShapeDtypeStruct((N_NODES, D), jnp.float32),
)

_tc_mid = pl.pallas_call(
    _tc_mid_body,
    grid=(GRID,),
    in_specs=[_deg_spec, _parts_spec, _row_spec, _vec_spec, _mat_spec],
    out_specs=_row_spec,
    out_shape=jax.ShapeDtypeStruct((N_NODES, D), jnp.float32),
)

_tc_post = pl.pallas_call(
    _tc_post_body,
    grid=(GRID,),
    in_specs=[_deg_spec, _parts_spec, _row_spec, _vec_spec],
    out_specs=_row_spec,
    out_shape=jax.ShapeDtypeStruct((N_NODES, D), jnp.float32),
)


def kernel(x, edge_index, W1, b1, W2, b2):
    src = edge_index[0].astype(jnp.int32)
    dst = edge_index[1].astype(jnp.int32)
    b1r = b1.reshape(1, D)
    b2r = b2.reshape(1, D)
    zeros = jnp.zeros((N_PAD, D), jnp.float32)

    degp = _sc_degree(dst, jnp.zeros((N_PAD,), jnp.float32))
    deg_parts = degp.reshape(NC, N_PAD)[:, :N_NODES].reshape(NC, N_NODES, 1)
    g1 = _tc_pre(deg_parts, x, W1)
    s1 = _sc_edge_agg(g1, src, dst, zeros)
    g2 = _tc_mid(deg_parts, s1, g1, b1r, W2)
    s2 = _sc_edge_agg(g2, src, dst, zeros)
    return _tc_post(deg_parts, s2, g2, b2r)


# speedup vs baseline: 3.2042x; 1.0329x over previous
import functools

import jax
import jax.numpy as jnp
from jax import lax
from jax.experimental import pallas as pl
from jax.experimental.pallas import tpu as pltpu
from jax.experimental.pallas import tpu_sc as plsc

N_NODES = 10000
N_EDGES = 320000
D = 128

NC = 2
NS = 16
NW = NC * NS
EW = N_EDGES // NW
CHUNK = 80
NCHUNK = EW // CHUNK
N_PAD = 10240
ROWS_PER_TILE = N_PAD // NS
NGROUP = (NCHUNK - 1) // 4

_mesh = plsc.VectorSubcoreMesh(core_axis_name="c", subcore_axis_name="s")


def _idx_load(idx_hbm, ebase, j, slot, sem):
    return pltpu.async_copy(idx_hbm.at[pl.ds(ebase + j * CHUNK, CHUNK)], slot, sem)


def _idx_wait(idx_hbm, ebase, j, slot, sem):
    pltpu.make_async_copy(idx_hbm.at[pl.ds(ebase + j * CHUNK, CHUNK)], slot, sem).wait()


DEG_R = ROWS_PER_TILE // D
NKVEC = EW // 16


@functools.partial(
    pl.kernel,
    out_type=jax.ShapeDtypeStruct((NC, NS, DEG_R, D), jnp.float32),
    mesh=_mesh,
    compiler_params=pltpu.CompilerParams(needs_layout_passes=False),
    scratch_types=[
        pltpu.VMEM((EW,), jnp.int32),
        pltpu.VMEM((N_PAD,), jnp.float32),
        pltpu.VMEM((NS, ROWS_PER_TILE), jnp.float32),
        pltpu.VMEM((DEG_R, D), jnp.float32),
        pltpu.VMEM_SHARED((NS, N_PAD), jnp.float32),
    ],
)
def _sc_degree(dst_hbm, zeros1_hbm, out_hbm,
               dst_v, hist_v, stripes_v, acc2_v, hists_sh):
    cid = lax.axis_index("c")
    sid = lax.axis_index("s")
    wid = sid * NC + cid
    base = sid * ROWS_PER_TILE

    pltpu.sync_copy(dst_hbm.at[pl.ds(wid * EW, EW)], dst_v)
    pltpu.sync_copy(zeros1_hbm, hist_v)

    def body(k, _):
        iv = dst_v[pl.ds(k * 16, 16)]
        plsc.addupdate_scatter(hist_v, [iv], jnp.ones((16,), jnp.float32))
        return 0

    lax.fori_loop(0, NKVEC, body, 0)
    pltpu.sync_copy(hist_v, hists_sh.at[sid])
    plsc.subcore_barrier()
    pltpu.sync_copy(hists_sh.at[:, pl.ds(base, ROWS_PER_TILE)], stripes_v)
    for r in range(DEG_R):
        for c in range(D // 16):
            sl = pl.ds(r * D + c * 16, 16)
            s = stripes_v[0, sl]
            for h in range(1, NS):
                s = s + stripes_v[h, sl]
            acc2_v[r, pl.ds(c * 16, 16)] = s
    pltpu.sync_copy(acc2_v, out_hbm.at[cid, sid])


NGROUP8 = 15


@functools.partial(
    pl.kernel,
    out_type=jax.ShapeDtypeStruct((NC, N_PAD, D), jnp.float32),
    mesh=_mesh,
    scratch_types=(
        [pltpu.VMEM((CHUNK,), jnp.int32)] * 8 +
        [pltpu.VMEM((CHUNK,), jnp.int32)] * 8 +
        [pltpu.VMEM((CHUNK, D), jnp.float32)] * 4 +
        [pltpu.VMEM_SHARED((N_PAD, D), jnp.float32),
         pltpu.SemaphoreType.DMA,
         pltpu.SemaphoreType.DMA,
         pltpu.SemaphoreType.DMA]
    ),
)
def _sc_edge_agg(g_hbm, src_hbm, dst_hbm, zeros_hbm, out_hbm, *refs):
    (s0, s1, s2, s3, s4, s5, s6, s7,
     d0, d1, d2, d3, d4, d5, d6, d7,
     r0, r1, r2, r3, acc_sh, isem, gsem, ssem) = refs
    cid = lax.axis_index("c")
    sid = lax.axis_index("s")
    wid = sid * NC + cid
    ebase = wid * EW
    base = sid * ROWS_PER_TILE
    sslot = [s0, s1, s2, s3, s4, s5, s6, s7]
    dslot = [d0, d1, d2, d3, d4, d5, d6, d7]
    rows = [r0, r1, r2, r3]

    def gather_start(j, r8, r4):
        pltpu.async_copy(g_hbm.at[sslot[r8]], rows[r4], gsem)

    def gather_wait(j, r8, r4):
        pltpu.make_async_copy(g_hbm.at[sslot[r8]], rows[r4], gsem).wait()

    def scatter_start(r8, r4):
        pltpu.async_copy(rows[r4], acc_sh.at[dslot[r8]], ssem, add=True)

    def scatter_drain(r8, r4):
        pltpu.make_async_copy(rows[r4], acc_sh.at[dslot[r8]], ssem).wait()

    def loads_start(j, r8):
        _idx_load(src_hbm, ebase, j, sslot[r8], isem)
        _idx_load(dst_hbm, ebase, j, dslot[r8], isem)

    def loads_wait(j, r8):
        _idx_wait(src_hbm, ebase, j, sslot[r8], isem)
        _idx_wait(dst_hbm, ebase, j, dslot[r8], isem)

    pltpu.sync_copy(zeros_hbm.at[pl.ds(base, ROWS_PER_TILE)],
                    acc_sh.at[pl.ds(base, ROWS_PER_TILE)])
    pltpu.sync_copy(src_hbm.at[pl.ds(ebase, CHUNK)], s0)
    pltpu.sync_copy(dst_hbm.at[pl.ds(ebase, CHUNK)], d0)
    loads_start(1, 1)
    loads_start(2, 2)
    plsc.subcore_barrier()

    gather_start(0, 0, 0)
    loads_wait(1, 1)
    gather_start(1, 1, 1)

    def body(g, _):
        for r in range(8):
            j = 8 * g + r

            @pl.when(j >= 2)
            def _():
                scatter_drain((r + 6) % 8, (r + 2) % 4)

            loads_wait(j + 2, (r + 2) % 8)
            gather_start(j + 2, (r + 2) % 8, (r + 2) % 4)
            loads_start(j + 3, (r + 3) % 8)
            gather_wait(j, r, r % 4)
            scatter_start(r, r % 4)
        return 0

    lax.fori_loop(0, NGROUP8, body, 0)
    scatter_drain(6, 2)
    loads_wait(122, 2)
    gather_start(122, 2, 2)
    loads_start(123, 3)
    gather_wait(120, 0, 0)
    scatter_start(0, 0)

    scatter_drain(7, 3)
    loads_wait(123, 3)
    gather_start(123, 3, 3)
    loads_start(124, 4)
    gather_wait(121, 1, 1)
    scatter_start(1, 1)

    scatter_drain(0, 0)
    loads_wait(124, 4)
    gather_start(124, 4, 0)
    gather_wait(122, 2, 2)
    scatter_start(2, 2)

    scatter_drain(1, 1)
    gather_wait(123, 3, 3)
    scatter_start(3, 3)

    scatter_drain(2, 2)
    gather_wait(124, 4, 0)
    scatter_start(4, 0)

    scatter_drain(3, 3)
    scatter_drain(4, 0)
    plsc.subcore_barrier()
    pltpu.sync_copy(acc_sh.at[pl.ds(base, ROWS_PER_TILE)],
                    out_hbm.at[cid, pl.ds(base, ROWS_PER_TILE)])


BR = 2000
GRID = N_NODES // BR


def _dinv_from_parts(deg_ref):
    deg = deg_ref[0, :, 0] + deg_ref[1, :, 0] + 1.0
    return lax.rsqrt(deg)[:, None]


def _tc_pre_body(deg_ref, x_ref, w_ref, g_ref):
    dinv = _dinv_from_parts(deg_ref)
    g_ref[...] = jnp.dot(x_ref[...], w_ref[...],
                         preferred_element_type=jnp.float32) * dinv


def _tc_mid_body(deg_ref, s_ref, g_ref, b_ref, w_ref, g2_ref):
    dinv = _dinv_from_parts(deg_ref)
    h = jnp.maximum(dinv * (s_ref[0] + s_ref[1] + g_ref[...]) + b_ref[...], 0.0)
    g2_ref[...] = jnp.dot(h, w_ref[...],
                          preferred_element_type=jnp.float32) * dinv


def _tc_post_body(deg_ref, s_ref, g_ref, b_ref, out_ref):
    dinv = _dinv_from_parts(deg_ref)
    out_ref[...] = dinv * (s_ref[0] + s_ref[1] + g_ref[...]) + b_ref[...]


_deg_spec = pl.BlockSpec((NC, BR, 1), lambda i: (0, i, 0))
_row_spec = pl.BlockSpec((BR, D), lambda i: (i, 0))
_parts_spec = pl.BlockSpec((NC, BR, D), lambda i: (0, i, 0))
_mat_spec = pl.BlockSpec((D, D), lambda i: (0, 0))
_vec_spec = pl.BlockSpec((1, D), lambda i: (0, 0))

_tc_pre = pl.pallas_call(
    _tc_pre_body,
    grid=(GRID,),
    in_specs=[_deg_spec, _row_spec, _mat_spec],
    out_specs=_row_spec,
    out_shape=jax.ShapeDtypeStruct((N_NODES, D), jnp.float32),
)

_tc_mid = pl.pallas_call(
    _tc_mid_body,
    grid=(GRID,),
    in_specs=[_deg_spec, _parts_spec, _row_spec, _vec_spec, _mat_spec],
    out_specs=_row_spec,
    out_shape=jax.ShapeDtypeStruct((N_NODES, D), jnp.float32),
)

_tc_post = pl.pallas_call(
    _tc_post_body,
    grid=(GRID,),
    in_specs=[_deg_spec, _parts_spec, _row_spec, _vec_spec],
    out_specs=_row_spec,
    out_shape=jax.ShapeDtypeStruct((N_NODES, D), jnp.float32),
)


def kernel(x, edge_index, W1, b1, W2, b2):
    src = edge_index[0].astype(jnp.int32)
    dst = edge_index[1].astype(jnp.int32)
    b1r = b1.reshape(1, D)
    b2r = b2.reshape(1, D)
    zeros = jnp.zeros((N_PAD, D), jnp.float32)

    degp = _sc_degree(dst, jnp.zeros((N_PAD,), jnp.float32))
    deg_parts = degp.reshape(NC, N_PAD)[:, :N_NODES].reshape(NC, N_NODES, 1)
    g1 = _tc_pre(deg_parts, x, W1)
    s1 = _sc_edge_agg(g1, src, dst, zeros)
    g2 = _tc_mid(deg_parts, s1, g1, b1r, W2)
    s2 = _sc_edge_agg(g2, src, dst, zeros)
    return _tc_post(deg_parts, s2, g2, b2r)
